# baseline (speedup 1.0000x reference)
import jax
import jax.numpy as jnp
from jax.experimental import pallas as pl

_BLK = 256


def _max_body(mail_ref, out_ref):
    out_ref[...] = jnp.max(mail_ref[...], axis=1)


def kernel(mailbox):
    n, m, d = mailbox.shape
    grid = (-(-n // _BLK),)
    return pl.pallas_call(
        _max_body,
        grid=grid,
        in_specs=[pl.BlockSpec((_BLK, m, d), lambda i: (i, 0, 0))],
        out_specs=pl.BlockSpec((_BLK, d), lambda i: (i, 0)),
        out_shape=jax.ShapeDtypeStruct((n, d), mailbox.dtype),
    )(mailbox)


# speedup vs baseline: 1.0979x; 1.0979x over previous
import jax
import jax.numpy as jnp
from jax.experimental import pallas as pl

_BLK = 768


def _max_body(mail_ref, out_ref):
    out_ref[...] = jnp.max(mail_ref[...], axis=1)


def kernel(mailbox):
    n, m, d = mailbox.shape
    grid = (-(-n // _BLK),)
    return pl.pallas_call(
        _max_body,
        grid=grid,
        in_specs=[pl.BlockSpec((_BLK, m, d), lambda i: (i, 0, 0))],
        out_specs=pl.BlockSpec((_BLK, d), lambda i: (i, 0)),
        out_shape=jax.ShapeDtypeStruct((n, d), mailbox.dtype),
    )(mailbox)


# speedup vs baseline: 1.1318x; 1.0309x over previous
import jax
import jax.numpy as jnp
from jax.experimental import pallas as pl

_BLK = 640


def _max_body(mail_ref, out_ref):
    out_ref[...] = jnp.max(mail_ref[...], axis=1)


def kernel(mailbox):
    n, m, d = mailbox.shape
    grid = (-(-n // _BLK),)
    return pl.pallas_call(
        _max_body,
        grid=grid,
        in_specs=[pl.BlockSpec((_BLK, m, d), lambda i: (i, 0, 0))],
        out_specs=pl.BlockSpec((_BLK, d), lambda i: (i, 0)),
        out_shape=jax.ShapeDtypeStruct((n, d), mailbox.dtype),
    )(mailbox)


# speedup vs baseline: 1.1344x; 1.0023x over previous
import jax
import jax.numpy as jnp
from jax.experimental import pallas as pl

_BLK = 480


def _max_body(mail_ref, out_ref):
    out_ref[...] = jnp.max(mail_ref[...], axis=1)


def kernel(mailbox):
    n, m, d = mailbox.shape
    grid = (-(-n // _BLK),)
    return pl.pallas_call(
        _max_body,
        grid=grid,
        in_specs=[pl.BlockSpec((_BLK, m, d), lambda i: (i, 0, 0))],
        out_specs=pl.BlockSpec((_BLK, d), lambda i: (i, 0)),
        out_shape=jax.ShapeDtypeStruct((n, d), mailbox.dtype),
    )(mailbox)
